# baseline (speedup 1.0000x reference)
import functools

import jax
import jax.numpy as jnp
from jax.experimental import pallas as pl
from jax.experimental.pallas import tpu as pltpu
from jax.experimental.pallas import tpu_sc as plsc

_WIN = 128


def _gather_rows(weight, token_ids, b, t, d):
    mesh = plsc.VectorSubcoreMesh(core_axis_name="core",
                                  subcore_axis_name="subcore")

    @functools.partial(
        pl.kernel,
        out_type=jax.ShapeDtypeStruct((b, t, d), weight.dtype),
        mesh=mesh,
        scratch_types=[pltpu.SemaphoreType.DMA],
        compiler_params=pltpu.CompilerParams(use_tc_tiling_on_sc=False),
    )
    def gather_kernel(w_hbm, i_hbm, o_hbm, sem):
        def body(i_vmem, o_vmem):
            copies = []
            base = 0
            while base < t:
                win = min(_WIN, t - base)
                copies.append(pltpu.async_copy(
                    w_hbm.at[i_vmem.at[0, pl.ds(base, win)]],
                    o_vmem.at[0, pl.ds(base, win)],
                    sem,
                ))
                base += win
            for c in copies:
                c.wait()

        pltpu.emit_pipeline(
            body,
            grid=(b,),
            in_specs=[pl.BlockSpec((1, t), index_map=lambda i: (i, 0))],
            out_specs=[pl.BlockSpec((1, t, d), index_map=lambda i: (i, 0, 0))],
            core_axis_name=("core", "subcore"),
            dimension_semantics=(pltpu.PARALLEL,),
        )(i_hbm, o_hbm)

    return gather_kernel(weight, token_ids)


def kernel(token_ids, weight):
    b, t = token_ids.shape
    d = weight.shape[1]
    return _gather_rows(weight, token_ids.astype(jnp.int32), b, t, d)


# speedup vs baseline: 1.3377x; 1.3377x over previous
import functools

import jax
import jax.numpy as jnp
from jax.experimental import pallas as pl
from jax.experimental.pallas import tpu as pltpu
from jax.experimental.pallas import tpu_sc as plsc

_WIN = 128
_PD = 128


def _gather_rows(w128, token_ids, b, t):
    mesh = plsc.VectorSubcoreMesh(core_axis_name="core",
                                  subcore_axis_name="subcore")

    @functools.partial(
        pl.kernel,
        out_type=jax.ShapeDtypeStruct((b, t, _PD), w128.dtype),
        mesh=mesh,
        scratch_types=[pltpu.SemaphoreType.DMA],
        compiler_params=pltpu.CompilerParams(use_tc_tiling_on_sc=False),
    )
    def gather_kernel(w_hbm, i_hbm, o_hbm, sem):
        def body(i_vmem, o_vmem):
            copies = []
            base = 0
            while base < t:
                win = min(_WIN, t - base)
                copies.append(pltpu.async_copy(
                    w_hbm.at[i_vmem.at[0, pl.ds(base, win)]],
                    o_vmem.at[0, pl.ds(base, win)],
                    sem,
                ))
                base += win
            for c in copies:
                c.wait()

        pltpu.emit_pipeline(
            body,
            grid=(b,),
            in_specs=[pl.BlockSpec((1, t), index_map=lambda i: (i, 0))],
            out_specs=[pl.BlockSpec((1, t, _PD),
                                    index_map=lambda i: (i, 0, 0))],
            core_axis_name=("core", "subcore"),
            dimension_semantics=(pltpu.PARALLEL,),
        )(i_hbm, o_hbm)

    return gather_kernel(w128, token_ids)


def kernel(token_ids, weight):
    b, t = token_ids.shape
    d = weight.shape[1]
    w128 = jnp.pad(weight, ((0, 0), (0, _PD - d)))
    out128 = _gather_rows(w128, token_ids.astype(jnp.int32), b, t)
    return out128[..., :d]


# speedup vs baseline: 1.5873x; 1.1866x over previous
import functools

import jax
import jax.numpy as jnp
from jax.experimental import pallas as pl
from jax.experimental.pallas import tpu as pltpu
from jax.experimental.pallas import tpu_sc as plsc

_WIN = 128
_PD = 128


def _gather_rows(weight, token_ids, b, t, d):
    mesh = plsc.VectorSubcoreMesh(core_axis_name="core",
                                  subcore_axis_name="subcore")

    @functools.partial(
        pl.kernel,
        out_type=jax.ShapeDtypeStruct((b, t, _PD), weight.dtype),
        mesh=mesh,
        scratch_types=[pltpu.SemaphoreType.DMA],
        compiler_params=pltpu.CompilerParams(use_tc_tiling_on_sc=False),
    )
    def gather_kernel(w_hbm, i_hbm, o_hbm, sem):
        def body(i_vmem, o_vmem):
            copies = []
            base = 0
            while base < t:
                win = min(_WIN, t - base)
                copies.append(pltpu.async_copy(
                    w_hbm.at[i_vmem.at[0, pl.ds(base, win)]],
                    o_vmem.at[0, pl.ds(base, win)],
                    sem,
                ))
                base += win
            for c in copies:
                c.wait()

        pltpu.emit_pipeline(
            body,
            grid=(b,),
            in_specs=[pl.BlockSpec((1, t), index_map=lambda i: (i, 0))],
            out_specs=[pl.BlockSpec((1, t, d),
                                    index_map=lambda i: (i, 0, 0))],
            core_axis_name=("core", "subcore"),
            dimension_semantics=(pltpu.PARALLEL,),
        )(i_hbm, o_hbm)

    return gather_kernel(weight, token_ids)


def kernel(token_ids, weight):
    b, t = token_ids.shape
    d = weight.shape[1]
    out128 = _gather_rows(weight, token_ids.astype(jnp.int32), b, t, d)
    return out128[..., :d]


# speedup vs baseline: 1.6321x; 1.0282x over previous
import functools

import jax
import jax.numpy as jnp
from jax.experimental import pallas as pl
from jax.experimental.pallas import tpu as pltpu
from jax.experimental.pallas import tpu_sc as plsc

_WIN = 128
_PD = 128


def _gather_rows(w2m, idx2, b, t, d):
    mesh = plsc.VectorSubcoreMesh(core_axis_name="core",
                                  subcore_axis_name="subcore")

    @functools.partial(
        pl.kernel,
        out_type=jax.ShapeDtypeStruct((b, t, _PD), w2m.dtype),
        mesh=mesh,
        scratch_types=[pltpu.SemaphoreType.DMA],
        compiler_params=pltpu.CompilerParams(use_tc_tiling_on_sc=False),
    )
    def gather_kernel(w_hbm, i_hbm, o_hbm, sem):
        def body(i_vmem, o_vmem):
            copies = []
            base = 0
            while base < t:
                win = min(_WIN, t - base)
                copies.append(pltpu.async_copy(
                    w_hbm.at[i_vmem.at[0, pl.ds(base, win)]],
                    o_vmem.at[0, pl.ds(base, win)],
                    sem,
                ))
                base += win
            for c in copies:
                c.wait()

        pltpu.emit_pipeline(
            body,
            grid=(b,),
            in_specs=[pl.BlockSpec((1, t), index_map=lambda i: (i, 0))],
            out_specs=[pl.BlockSpec((1, t, d),
                                    index_map=lambda i: (i, 0, 0))],
            core_axis_name=("core", "subcore"),
            dimension_semantics=(pltpu.PARALLEL,),
        )(i_hbm, o_hbm)

    return gather_kernel(w2m, idx2)


def kernel(token_ids, weight):
    b, t = token_ids.shape
    d = weight.shape[1]
    w128 = jnp.pad(weight, ((0, 0), (0, _PD - d)))
    w2m = w128.reshape(2 * weight.shape[0], d)
    idx2 = token_ids.astype(jnp.int32) * 2
    out128 = _gather_rows(w2m, idx2, b, t, d)
    return out128[..., :d]


# speedup vs baseline: 1.8072x; 1.1073x over previous
import functools

import jax
import jax.numpy as jnp
from jax.experimental import pallas as pl
from jax.experimental.pallas import tpu as pltpu
from jax.experimental.pallas import tpu_sc as plsc

_WIN = 128
_PD = 128
_RPS = 4


def _gather_rows(w2m, idx2, b, t, d):
    mesh = plsc.VectorSubcoreMesh(core_axis_name="core",
                                  subcore_axis_name="subcore")

    @functools.partial(
        pl.kernel,
        out_type=jax.ShapeDtypeStruct((b, t, _PD), w2m.dtype),
        mesh=mesh,
        scratch_types=[pltpu.SemaphoreType.DMA],
        compiler_params=pltpu.CompilerParams(use_tc_tiling_on_sc=False),
    )
    def gather_kernel(w_hbm, i_hbm, o_hbm, sem):
        def body(i_vmem, o_vmem):
            copies = []
            for r in range(_RPS):
                base = 0
                while base < t:
                    win = min(_WIN, t - base)
                    copies.append(pltpu.async_copy(
                        w_hbm.at[i_vmem.at[r, pl.ds(base, win)]],
                        o_vmem.at[r, pl.ds(base, win)],
                        sem,
                    ))
                    base += win
            for c in copies:
                c.wait()

        pltpu.emit_pipeline(
            body,
            grid=(b // _RPS,),
            in_specs=[pl.BlockSpec((_RPS, t), index_map=lambda i: (i, 0))],
            out_specs=[pl.BlockSpec((_RPS, t, d),
                                    index_map=lambda i: (i, 0, 0))],
            core_axis_name=("core", "subcore"),
            dimension_semantics=(pltpu.PARALLEL,),
        )(i_hbm, o_hbm)

    return gather_kernel(w2m, idx2)


def kernel(token_ids, weight):
    b, t = token_ids.shape
    d = weight.shape[1]
    w128 = jnp.pad(weight, ((0, 0), (0, _PD - d)))
    w2m = w128.reshape(2 * weight.shape[0], d)
    idx2 = token_ids.astype(jnp.int32) * 2
    out128 = _gather_rows(w2m, idx2, b, t, d)
    return out128[..., :d]
